# Initial kernel scaffold; baseline (speedup 1.0000x reference)
import jax
import jax.numpy as jnp
from jax.experimental import pallas as pl


def kernel(head_node, objective_nodes, value_nodes, edge_indices, W_head, b_head, W_obj, b_obj, W_val, b_val, W0, att_src0, att_dst0, bias0, W1, att_src1, att_dst1, bias1):
    raise NotImplementedError("write your pallas kernel here")


# speedup vs baseline: 55.9145x; 55.9145x over previous
import functools

import jax
import jax.numpy as jnp
import numpy as np
from jax.experimental import pallas as pl
from jax.experimental.pallas import tpu as pltpu

B = 4096
N_OBJ = 10
N_VAL = 50
N_PER = 61
NP = 64
E = 128
H = 128
HEADS = 4
DH = H // HEADS
G = 8

_NEG = np.float32(-1e30)


def _gat_block(xp2, av2, A_list, den_sel, bias):
    xp3 = xp2.reshape(G, NP, H)
    av3 = av2.reshape(G, NP, 2 * HEADS)
    outs = []
    for g in range(G):
        xp = xp3[g]
        asrc = av3[g][:, :HEADS]
        adst = av3[g][:, HEADS:]
        asrcT = jnp.transpose(asrc)
        alpha = adst[:, :, None] + asrcT[None, :, :]
        alpha = jnp.where(alpha > 0, alpha, 0.2 * alpha)
        Ag = A_list[g]
        mask = (Ag > 0.0)[:, None, :]
        alpha = jnp.where(mask, alpha, _NEG)
        amax = jnp.max(alpha, axis=2, keepdims=True)
        Ew = jnp.exp(alpha - amax) * Ag[:, None, :]
        den = jnp.sum(Ew, axis=2)
        pieces = []
        for h in range(HEADS):
            Eh = Ew[:, h, :]
            ph = jax.lax.dot(Eh, xp[:, h * DH:(h + 1) * DH])
            pieces.append(ph)
        og = jnp.concatenate(pieces, axis=1)
        deng = jax.lax.dot(den, den_sel)
        outs.append(og / deng + bias)
    return jnp.concatenate(outs, axis=0)


def _fused_kernel(feat_ref, ei_ref, wcat_ref, brow_ref,
                  w0_ref, att0_ref, b0_ref, w1_ref, att1_ref, b1_ref,
                  outh_ref, outv_ref):
    feat2 = feat_ref[...].reshape(G * NP, 9)
    x2 = jnp.maximum(jax.lax.dot(feat2, wcat_ref[...]), 0.0)
    x2 = (x2.reshape(G, NP, H) + brow_ref[...][None]).reshape(G * NP, H)
    x2 = jnp.maximum(x2, 0.0)

    lane_iota = jax.lax.broadcasted_iota(jnp.int32, (1, NP), 1)
    eye = jnp.float32(
        jax.lax.broadcasted_iota(jnp.int32, (NP, NP), 0)
        == jax.lax.broadcasted_iota(jnp.int32, (NP, NP), 1))

    A_list = []
    for g in range(G):
        src = ei_ref[g][:, 0:1]
        dst = ei_ref[g][:, 1:2]
        oh_src = jnp.float32(src == lane_iota)
        oh_dst = jnp.float32(dst == lane_iota)
        Ag = jax.lax.dot_general(
            oh_dst, oh_src, (((0,), (0,)), ((), ())))
        A_list.append(Ag + eye)

    hh = jax.lax.broadcasted_iota(jnp.int32, (HEADS, H), 0)
    cc = jax.lax.broadcasted_iota(jnp.int32, (HEADS, H), 1)
    den_sel = jnp.float32(hh == (cc // DH))

    def layer(x2, w_ref, att_ref, b_ref):
        xp2 = jax.lax.dot(x2, w_ref[...])
        av2 = jax.lax.dot(xp2, att_ref[...])
        return _gat_block(xp2, av2, A_list, den_sel, b_ref[...])

    h1 = jnp.maximum(layer(x2, w0_ref, att0_ref, b0_ref), 0.0)
    h2 = jnp.maximum(layer(h1, w1_ref, att1_ref, b1_ref), 0.0)
    h3 = h2.reshape(G, NP, H)
    outh_ref[...] = h3[:, 0, :]
    outv_ref[...] = h3[:, N_PER - N_VAL:N_PER, :]


@jax.jit
def _run(feat, ei_t, wcat, brow, w0, att0, b0, w1, att1, b1):
    grid = (B // G,)
    return pl.pallas_call(
        _fused_kernel,
        grid=grid,
        in_specs=[
            pl.BlockSpec((G, NP, 9), lambda i: (i, 0, 0)),
            pl.BlockSpec((G, E, 2), lambda i: (i, 0, 0)),
            pl.BlockSpec((9, H), lambda i: (0, 0)),
            pl.BlockSpec((NP, H), lambda i: (0, 0)),
            pl.BlockSpec((H, H), lambda i: (0, 0)),
            pl.BlockSpec((H, 2 * HEADS), lambda i: (0, 0)),
            pl.BlockSpec((1, H), lambda i: (0, 0)),
            pl.BlockSpec((H, H), lambda i: (0, 0)),
            pl.BlockSpec((H, 2 * HEADS), lambda i: (0, 0)),
            pl.BlockSpec((1, H), lambda i: (0, 0)),
        ],
        out_specs=[
            pl.BlockSpec((G, H), lambda i: (i, 0)),
            pl.BlockSpec((G, N_VAL, H), lambda i: (i, 0, 0)),
        ],
        out_shape=[
            jax.ShapeDtypeStruct((B, H), jnp.float32),
            jax.ShapeDtypeStruct((B, N_VAL, H), jnp.float32),
        ],
    )(feat, ei_t, wcat, brow, w0, att0, b0, w1, att1, b1)


def kernel(head_node, objective_nodes, value_nodes, edge_indices,
           W_head, b_head, W_obj, b_obj, W_val, b_val,
           W0, att_src0, att_dst0, bias0,
           W1, att_src1, att_dst1, bias1):
    f32 = jnp.float32
    feat = jnp.zeros((B, NP, 9), f32)
    feat = feat.at[:, 0, 0:2].set(head_node)
    feat = feat.at[:, 1:1 + N_OBJ, 2:4].set(objective_nodes)
    feat = feat.at[:, 1 + N_OBJ:N_PER, 4:9].set(value_nodes)
    wcat = jnp.concatenate([W_head, W_obj, W_val], axis=0)
    brow = jnp.concatenate([
        b_head[None, :],
        jnp.tile(b_obj[None, :], (N_OBJ, 1)),
        jnp.tile(b_val[None, :], (N_VAL, 1)),
        jnp.zeros((NP - N_PER, H), f32),
    ], axis=0)
    ei_t = jnp.transpose(edge_indices, (0, 2, 1))
    att0 = jnp.zeros((H, 2 * HEADS), f32)
    att1 = jnp.zeros((H, 2 * HEADS), f32)
    for h in range(HEADS):
        att0 = att0.at[h * DH:(h + 1) * DH, h].set(att_src0[h])
        att0 = att0.at[h * DH:(h + 1) * DH, HEADS + h].set(att_dst0[h])
        att1 = att1.at[h * DH:(h + 1) * DH, h].set(att_src1[h])
        att1 = att1.at[h * DH:(h + 1) * DH, HEADS + h].set(att_dst1[h])
    outh, outv = _run(feat, ei_t, wcat, brow,
                      W0, att0, bias0[None, :], W1, att1, bias1[None, :])
    return (outh, outv)


# speedup vs baseline: 77.0439x; 1.3779x over previous
import functools

import jax
import jax.numpy as jnp
import numpy as np
from jax.experimental import pallas as pl
from jax.experimental.pallas import tpu as pltpu

B = 4096
N_OBJ = 10
N_VAL = 50
N_PER = 61
NP = 64
E = 128
H = 128
HEADS = 4
DH = H // HEADS
HS = HEADS * NP
G = 8


def _leaky(x):
    return jnp.maximum(x, 0.2 * x)


def _gat_block(xp2, av2, A4_list, sel_den, rexp, bias):
    asrc3 = av2[:, :HEADS].reshape(G, NP, HEADS)
    adst2 = av2[:, HEADS:]
    u = jnp.transpose(asrc3, (0, 2, 1)).reshape(G, 1, HS)
    u_all = jnp.broadcast_to(u, (G, NP, HS)).reshape(G * NP, HS)
    gmax = jnp.max(asrc3, axis=1, keepdims=True)
    gmax_all = jnp.broadcast_to(gmax, (G, NP, HEADS)).reshape(G * NP, HEADS)
    shift = _leaky(adst2 + gmax_all)
    alpha = _leaky(jax.lax.dot(adst2, rexp) + u_all)
    ex = jnp.exp(alpha - jax.lax.dot(shift, rexp))

    xp3 = xp2.reshape(G, NP, H)
    ex3 = ex.reshape(G, NP, HS)
    outs = []
    for g in range(G):
        Eg = ex3[g] * A4_list[g]
        deng = jax.lax.dot(Eg, sel_den)
        xp = xp3[g]
        pieces = []
        for h in range(HEADS):
            Eh = Eg[:, h * NP:(h + 1) * NP]
            pieces.append(jax.lax.dot(Eh, xp[:, h * DH:(h + 1) * DH]))
        og = jnp.concatenate(pieces, axis=1)
        outs.append(og / deng + bias)
    return jnp.concatenate(outs, axis=0)


def _fused_kernel(feat_ref, ei_ref, wcat_ref, brow_ref,
                  w0_ref, att0_ref, b0_ref, w1_ref, att1_ref, b1_ref,
                  outh_ref, outv_ref):
    feat2 = feat_ref[...].reshape(G * NP, 9)
    x2 = jnp.maximum(jax.lax.dot(feat2, wcat_ref[...]), 0.0)
    x2 = (x2.reshape(G, NP, H) + brow_ref[...][None]).reshape(G * NP, H)
    x2 = jnp.maximum(x2, 0.0)

    lane_np = jax.lax.broadcasted_iota(jnp.int32, (1, NP), 1)
    lane_hs = jax.lax.broadcasted_iota(jnp.int32, (1, HS), 1) % NP
    d_iota = jax.lax.broadcasted_iota(jnp.int32, (NP, HS), 0)
    s_iota = jax.lax.broadcasted_iota(jnp.int32, (NP, HS), 1) % NP
    eye4 = jnp.float32(d_iota == s_iota)

    A4_list = []
    for g in range(G):
        src = ei_ref[g][:, 0:1]
        dst = ei_ref[g][:, 1:2]
        oh_src4 = jnp.float32(src == lane_hs)
        oh_dst = jnp.float32(dst == lane_np)
        A4 = jax.lax.dot_general(
            oh_dst, oh_src4, (((0,), (0,)), ((), ())))
        A4_list.append(A4 + eye4)

    hs_row = jax.lax.broadcasted_iota(jnp.int32, (HS, H), 0) // NP
    cc = jax.lax.broadcasted_iota(jnp.int32, (HS, H), 1) // DH
    sel_den = jnp.float32(hs_row == cc)
    rh = jax.lax.broadcasted_iota(jnp.int32, (HEADS, HS), 0)
    rc = jax.lax.broadcasted_iota(jnp.int32, (HEADS, HS), 1) // NP
    rexp = jnp.float32(rh == rc)

    def layer(x2, w_ref, att_ref, b_ref):
        xp2 = jax.lax.dot(x2, w_ref[...])
        av2 = jax.lax.dot(xp2, att_ref[...])
        return _gat_block(xp2, av2, A4_list, sel_den, rexp, b_ref[...])

    h1 = jnp.maximum(layer(x2, w0_ref, att0_ref, b0_ref), 0.0)
    h2 = jnp.maximum(layer(h1, w1_ref, att1_ref, b1_ref), 0.0)
    h3 = h2.reshape(G, NP, H)
    outh_ref[...] = h3[:, 0, :]
    outv_ref[...] = h3[:, N_PER - N_VAL:N_PER, :]


@jax.jit
def _run(feat, ei_t, wcat, brow, w0, att0, b0, w1, att1, b1):
    grid = (B // G,)
    return pl.pallas_call(
        _fused_kernel,
        grid=grid,
        in_specs=[
            pl.BlockSpec((G, NP, 9), lambda i: (i, 0, 0)),
            pl.BlockSpec((G, E, 2), lambda i: (i, 0, 0)),
            pl.BlockSpec((9, H), lambda i: (0, 0)),
            pl.BlockSpec((NP, H), lambda i: (0, 0)),
            pl.BlockSpec((H, H), lambda i: (0, 0)),
            pl.BlockSpec((H, 2 * HEADS), lambda i: (0, 0)),
            pl.BlockSpec((1, H), lambda i: (0, 0)),
            pl.BlockSpec((H, H), lambda i: (0, 0)),
            pl.BlockSpec((H, 2 * HEADS), lambda i: (0, 0)),
            pl.BlockSpec((1, H), lambda i: (0, 0)),
        ],
        out_specs=[
            pl.BlockSpec((G, H), lambda i: (i, 0)),
            pl.BlockSpec((G, N_VAL, H), lambda i: (i, 0, 0)),
        ],
        out_shape=[
            jax.ShapeDtypeStruct((B, H), jnp.float32),
            jax.ShapeDtypeStruct((B, N_VAL, H), jnp.float32),
        ],
    )(feat, ei_t, wcat, brow, w0, att0, b0, w1, att1, b1)


def kernel(head_node, objective_nodes, value_nodes, edge_indices,
           W_head, b_head, W_obj, b_obj, W_val, b_val,
           W0, att_src0, att_dst0, bias0,
           W1, att_src1, att_dst1, bias1):
    f32 = jnp.float32
    feat = jnp.zeros((B, NP, 9), f32)
    feat = feat.at[:, 0, 0:2].set(head_node)
    feat = feat.at[:, 1:1 + N_OBJ, 2:4].set(objective_nodes)
    feat = feat.at[:, 1 + N_OBJ:N_PER, 4:9].set(value_nodes)
    wcat = jnp.concatenate([W_head, W_obj, W_val], axis=0)
    brow = jnp.concatenate([
        b_head[None, :],
        jnp.tile(b_obj[None, :], (N_OBJ, 1)),
        jnp.tile(b_val[None, :], (N_VAL, 1)),
        jnp.zeros((NP - N_PER, H), f32),
    ], axis=0)
    ei_t = jnp.transpose(edge_indices, (0, 2, 1))
    att0 = jnp.zeros((H, 2 * HEADS), f32)
    att1 = jnp.zeros((H, 2 * HEADS), f32)
    for h in range(HEADS):
        att0 = att0.at[h * DH:(h + 1) * DH, h].set(att_src0[h])
        att0 = att0.at[h * DH:(h + 1) * DH, HEADS + h].set(att_dst0[h])
        att1 = att1.at[h * DH:(h + 1) * DH, h].set(att_src1[h])
        att1 = att1.at[h * DH:(h + 1) * DH, HEADS + h].set(att_dst1[h])
    outh, outv = _run(feat, ei_t, wcat, brow,
                      W0, att0, bias0[None, :], W1, att1, bias1[None, :])
    return (outh, outv)


# speedup vs baseline: 330.4431x; 4.2890x over previous
import functools

import jax
import jax.numpy as jnp
import numpy as np
from jax.experimental import pallas as pl
from jax.experimental.pallas import tpu as pltpu

B = 4096
N_OBJ = 10
N_VAL = 50
N_PER = 61
NP = 64
E = 128
H = 128
HEADS = 4
DH = H // HEADS
HS = HEADS * NP
G = 8


def _leaky(x):
    return jnp.maximum(x, 0.2 * x)


def _gat_block(xp2, av2, A4_list, sel_den, rexp, bias):
    asrc3 = av2[:, :HEADS].reshape(G, NP, HEADS)
    adst2 = av2[:, HEADS:]
    u = jnp.transpose(asrc3, (0, 2, 1)).reshape(G, 1, HS)
    u_all = jnp.broadcast_to(u, (G, NP, HS)).reshape(G * NP, HS)
    gmax = jnp.max(asrc3, axis=1, keepdims=True)
    gmax_all = jnp.broadcast_to(gmax, (G, NP, HEADS)).reshape(G * NP, HEADS)
    shift = _leaky(adst2 + gmax_all)
    alpha = _leaky(jax.lax.dot(adst2, rexp) + u_all)
    ex = jnp.exp(alpha - jax.lax.dot(shift, rexp))

    xp3 = xp2.reshape(G, NP, H)
    ex3 = ex.reshape(G, NP, HS)
    outs = []
    for g in range(G):
        Eg = ex3[g] * A4_list[g]
        deng = jax.lax.dot(Eg, sel_den)
        xp = xp3[g]
        pieces = []
        for h in range(HEADS):
            Eh = Eg[:, h * NP:(h + 1) * NP]
            pieces.append(jax.lax.dot(Eh, xp[:, h * DH:(h + 1) * DH]))
        og = jnp.concatenate(pieces, axis=1)
        outs.append(og / deng + bias)
    return jnp.concatenate(outs, axis=0)


def _fused_kernel(feat_ref, ei_ref, wcat_ref, brow_ref,
                  w0_ref, att0_ref, b0_ref, w1_ref, att1_ref, b1_ref,
                  outh_ref, outv_ref):
    x2 = jax.lax.dot_general(
        feat_ref[...], wcat_ref[...], (((0,), (0,)), ((), ())))
    x2 = (x2.reshape(G, NP, H) + brow_ref[...][None]).reshape(G * NP, H)
    x2 = jnp.maximum(x2, 0.0)

    col_np = jax.lax.broadcasted_iota(jnp.int32, (NP, 1), 0)
    col_hs = jax.lax.broadcasted_iota(jnp.int32, (HS, 1), 0) % NP
    d_iota = jax.lax.broadcasted_iota(jnp.int32, (NP, HS), 0)
    s_iota = jax.lax.broadcasted_iota(jnp.int32, (NP, HS), 1) % NP
    eye4 = jnp.float32(d_iota == s_iota)

    A4_list = []
    for g in range(G):
        src = ei_ref[g][0:1, :]
        dst = ei_ref[g][1:2, :]
        oh_src4T = jnp.float32(src == col_hs)
        oh_dstT = jnp.float32(dst == col_np)
        A4 = jax.lax.dot_general(
            oh_dstT, oh_src4T, (((1,), (1,)), ((), ())))
        A4_list.append(A4 + eye4)

    hs_row = jax.lax.broadcasted_iota(jnp.int32, (HS, H), 0) // NP
    cc = jax.lax.broadcasted_iota(jnp.int32, (HS, H), 1) // DH
    sel_den = jnp.float32(hs_row == cc)
    rh = jax.lax.broadcasted_iota(jnp.int32, (HEADS, HS), 0)
    rc = jax.lax.broadcasted_iota(jnp.int32, (HEADS, HS), 1) // NP
    rexp = jnp.float32(rh == rc)

    def layer(x2, w_ref, att_ref, b_ref):
        xp2 = jax.lax.dot(x2, w_ref[...])
        av2 = jax.lax.dot(xp2, att_ref[...])
        return _gat_block(xp2, av2, A4_list, sel_den, rexp, b_ref[...])

    h1 = jnp.maximum(layer(x2, w0_ref, att0_ref, b0_ref), 0.0)
    h2 = jnp.maximum(layer(h1, w1_ref, att1_ref, b1_ref), 0.0)
    h3 = h2.reshape(G, NP, H)
    outh_ref[...] = h3[:, 0, :]
    outv_ref[...] = h3[:, N_PER - N_VAL:N_PER, :]


@jax.jit
def _run(feat, ei_t, wcat, brow, w0, att0, b0, w1, att1, b1):
    grid = (B // G,)
    return pl.pallas_call(
        _fused_kernel,
        grid=grid,
        in_specs=[
            pl.BlockSpec((9, G * NP), lambda i: (0, i)),
            pl.BlockSpec((G, 2, E), lambda i: (i, 0, 0)),
            pl.BlockSpec((9, H), lambda i: (0, 0)),
            pl.BlockSpec((NP, H), lambda i: (0, 0)),
            pl.BlockSpec((H, H), lambda i: (0, 0)),
            pl.BlockSpec((H, 2 * HEADS), lambda i: (0, 0)),
            pl.BlockSpec((1, H), lambda i: (0, 0)),
            pl.BlockSpec((H, H), lambda i: (0, 0)),
            pl.BlockSpec((H, 2 * HEADS), lambda i: (0, 0)),
            pl.BlockSpec((1, H), lambda i: (0, 0)),
        ],
        out_specs=[
            pl.BlockSpec((G, H), lambda i: (i, 0)),
            pl.BlockSpec((G, N_VAL, H), lambda i: (i, 0, 0)),
        ],
        out_shape=[
            jax.ShapeDtypeStruct((B, H), jnp.float32),
            jax.ShapeDtypeStruct((B, N_VAL, H), jnp.float32),
        ],
    )(feat, ei_t, wcat, brow, w0, att0, b0, w1, att1, b1)


def kernel(head_node, objective_nodes, value_nodes, edge_indices,
           W_head, b_head, W_obj, b_obj, W_val, b_val,
           W0, att_src0, att_dst0, bias0,
           W1, att_src1, att_dst1, bias1):
    f32 = jnp.float32
    feat3 = jnp.zeros((9, B, NP), f32)
    feat3 = feat3.at[0:2, :, 0].set(jnp.transpose(head_node))
    feat3 = feat3.at[2:4, :, 1:1 + N_OBJ].set(
        jnp.transpose(objective_nodes, (2, 0, 1)))
    feat3 = feat3.at[4:9, :, 1 + N_OBJ:N_PER].set(
        jnp.transpose(value_nodes, (2, 0, 1)))
    feat = feat3.reshape(9, B * NP)
    wcat = jnp.concatenate([W_head, W_obj, W_val], axis=0)
    brow = jnp.concatenate([
        b_head[None, :],
        jnp.tile(b_obj[None, :], (N_OBJ, 1)),
        jnp.tile(b_val[None, :], (N_VAL, 1)),
        jnp.zeros((NP - N_PER, H), f32),
    ], axis=0)
    att0 = jnp.zeros((H, 2 * HEADS), f32)
    att1 = jnp.zeros((H, 2 * HEADS), f32)
    for h in range(HEADS):
        att0 = att0.at[h * DH:(h + 1) * DH, h].set(att_src0[h])
        att0 = att0.at[h * DH:(h + 1) * DH, HEADS + h].set(att_dst0[h])
        att1 = att1.at[h * DH:(h + 1) * DH, h].set(att_src1[h])
        att1 = att1.at[h * DH:(h + 1) * DH, HEADS + h].set(att_dst1[h])
    outh, outv = _run(feat, edge_indices, wcat, brow,
                      W0, att0, bias0[None, :], W1, att1, bias1[None, :])
    return (outh, outv)


# speedup vs baseline: 412.9557x; 1.2497x over previous
import functools

import jax
import jax.numpy as jnp
import numpy as np
from jax.experimental import pallas as pl
from jax.experimental.pallas import tpu as pltpu

B = 4096
N_OBJ = 10
N_VAL = 50
N_PER = 61
NP = 64
E = 128
H = 128
HEADS = 4
DH = H // HEADS
HS = HEADS * NP
G = 8


def _leaky(x):
    return jnp.maximum(x, 0.2 * x)


def _gat_block(xp2, av2, A4_list, sel_den, rexp2, colmask, bias):
    asrc3 = av2[:, :HEADS].reshape(G, NP, HEADS)
    adst2 = av2[:, HEADS:]
    u = jnp.transpose(asrc3, (0, 2, 1)).reshape(G, 1, HS)
    u_all = jnp.broadcast_to(u, (G, NP, HS)).reshape(G * NP, HS)
    gmax = jnp.max(asrc3, axis=1, keepdims=True)
    gmax_all = jnp.broadcast_to(gmax, (G, NP, HEADS)).reshape(G * NP, HEADS)
    shift = _leaky(adst2 + gmax_all)
    rd = jax.lax.dot(jnp.concatenate([adst2, shift], axis=1), rexp2)
    ex = jnp.exp(_leaky(rd[:, :HS] + u_all) - rd[:, HS:])

    xp3 = xp2.reshape(G, NP, H)
    ex3 = ex.reshape(G, NP, HS)
    outs = []
    for g in range(G):
        Eg = ex3[g] * A4_list[g]
        xstk = (xp3[g][None] * colmask).reshape(HS, H)
        res = jax.lax.dot(Eg, jnp.concatenate([xstk, sel_den], axis=1))
        outs.append(res[:, :H] / res[:, H:] + bias)
    return jnp.concatenate(outs, axis=0)


def _fused_kernel(feat_ref, ei_ref, wcat_ref, brow_ref,
                  w0_ref, att0_ref, b0_ref, w1_ref, att1_ref, b1_ref,
                  outh_ref, outv_ref):
    x2 = jax.lax.dot_general(
        feat_ref[...], wcat_ref[...], (((0,), (0,)), ((), ())))
    x2 = (x2.reshape(G, NP, H) + brow_ref[...][None]).reshape(G * NP, H)
    x2 = jnp.maximum(x2, 0.0)

    col_np = jax.lax.broadcasted_iota(jnp.int32, (NP, 1), 0)
    col_hs = jax.lax.broadcasted_iota(jnp.int32, (HS, 1), 0) % NP
    d_iota = jax.lax.broadcasted_iota(jnp.int32, (NP, HS), 0)
    s_iota = jax.lax.broadcasted_iota(jnp.int32, (NP, HS), 1) % NP
    eye4 = jnp.float32(d_iota == s_iota)

    A4_list = []
    for g in range(G):
        src = ei_ref[g][0:1, :]
        dst = ei_ref[g][1:2, :]
        oh_src4T = jnp.float32(src == col_hs)
        oh_dstT = jnp.float32(dst == col_np)
        A4 = jax.lax.dot_general(
            oh_dstT, oh_src4T, (((1,), (1,)), ((), ())))
        A4_list.append(A4 + eye4)

    hs_row = jax.lax.broadcasted_iota(jnp.int32, (HS, H), 0) // NP
    cc = jax.lax.broadcasted_iota(jnp.int32, (HS, H), 1) // DH
    sel_den = jnp.float32(hs_row == cc)
    rh = jax.lax.broadcasted_iota(jnp.int32, (2 * HEADS, 2 * HS), 0)
    rc = jax.lax.broadcasted_iota(jnp.int32, (2 * HEADS, 2 * HS), 1) // NP
    rexp2 = jnp.float32(rh == rc)
    mh = jax.lax.broadcasted_iota(jnp.int32, (HEADS, 1, H), 0)
    mc = jax.lax.broadcasted_iota(jnp.int32, (HEADS, 1, H), 2) // DH
    colmask = jnp.float32(mh == mc)

    def layer(x2, w_ref, att_ref, b_ref):
        xp2 = jax.lax.dot(x2, w_ref[...])
        av2 = jax.lax.dot(xp2, att_ref[...])
        return _gat_block(xp2, av2, A4_list, sel_den, rexp2, colmask,
                          b_ref[...])

    h1 = jnp.maximum(layer(x2, w0_ref, att0_ref, b0_ref), 0.0)
    h2 = jnp.maximum(layer(h1, w1_ref, att1_ref, b1_ref), 0.0)
    h3 = h2.reshape(G, NP, H)
    outh_ref[...] = h3[:, 0, :]
    outv_ref[...] = h3[:, N_PER - N_VAL:N_PER, :]


@jax.jit
def _run(feat, ei_t, wcat, brow, w0, att0, b0, w1, att1, b1):
    grid = (B // G,)
    return pl.pallas_call(
        _fused_kernel,
        grid=grid,
        in_specs=[
            pl.BlockSpec((9, G * NP), lambda i: (0, i)),
            pl.BlockSpec((G, 2, E), lambda i: (i, 0, 0)),
            pl.BlockSpec((9, H), lambda i: (0, 0)),
            pl.BlockSpec((NP, H), lambda i: (0, 0)),
            pl.BlockSpec((H, H), lambda i: (0, 0)),
            pl.BlockSpec((H, 2 * HEADS), lambda i: (0, 0)),
            pl.BlockSpec((1, H), lambda i: (0, 0)),
            pl.BlockSpec((H, H), lambda i: (0, 0)),
            pl.BlockSpec((H, 2 * HEADS), lambda i: (0, 0)),
            pl.BlockSpec((1, H), lambda i: (0, 0)),
        ],
        out_specs=[
            pl.BlockSpec((G, H), lambda i: (i, 0)),
            pl.BlockSpec((G, N_VAL, H), lambda i: (i, 0, 0)),
        ],
        out_shape=[
            jax.ShapeDtypeStruct((B, H), jnp.float32),
            jax.ShapeDtypeStruct((B, N_VAL, H), jnp.float32),
        ],
    )(feat, ei_t, wcat, brow, w0, att0, b0, w1, att1, b1)


def kernel(head_node, objective_nodes, value_nodes, edge_indices,
           W_head, b_head, W_obj, b_obj, W_val, b_val,
           W0, att_src0, att_dst0, bias0,
           W1, att_src1, att_dst1, bias1):
    f32 = jnp.float32
    feat3 = jnp.zeros((9, B, NP), f32)
    feat3 = feat3.at[0:2, :, 0].set(jnp.transpose(head_node))
    feat3 = feat3.at[2:4, :, 1:1 + N_OBJ].set(
        jnp.transpose(objective_nodes, (2, 0, 1)))
    feat3 = feat3.at[4:9, :, 1 + N_OBJ:N_PER].set(
        jnp.transpose(value_nodes, (2, 0, 1)))
    feat = feat3.reshape(9, B * NP)
    wcat = jnp.concatenate([W_head, W_obj, W_val], axis=0)
    brow = jnp.concatenate([
        b_head[None, :],
        jnp.tile(b_obj[None, :], (N_OBJ, 1)),
        jnp.tile(b_val[None, :], (N_VAL, 1)),
        jnp.zeros((NP - N_PER, H), f32),
    ], axis=0)
    att0 = jnp.zeros((H, 2 * HEADS), f32)
    att1 = jnp.zeros((H, 2 * HEADS), f32)
    for h in range(HEADS):
        att0 = att0.at[h * DH:(h + 1) * DH, h].set(att_src0[h])
        att0 = att0.at[h * DH:(h + 1) * DH, HEADS + h].set(att_dst0[h])
        att1 = att1.at[h * DH:(h + 1) * DH, h].set(att_src1[h])
        att1 = att1.at[h * DH:(h + 1) * DH, HEADS + h].set(att_dst1[h])
    outh, outv = _run(feat, edge_indices, wcat, brow,
                      W0, att0, bias0[None, :], W1, att1, bias1[None, :])
    return (outh, outv)


# speedup vs baseline: 499.0982x; 1.2086x over previous
import functools

import jax
import jax.numpy as jnp
import numpy as np
from jax.experimental import pallas as pl
from jax.experimental.pallas import tpu as pltpu

B = 4096
N_OBJ = 10
N_VAL = 50
N_PER = 61
NP = 64
E = 128
H = 128
HEADS = 4
DH = H // HEADS
HS = HEADS * NP
G = 16


def _leaky(x):
    return jnp.maximum(x, 0.2 * x)


def _gat_block(xp2, av2, A4_list, sel_den, rexp, gh1, colmask, bias):
    asrc3 = av2[:, :HEADS].reshape(G, NP, HEADS)
    adst2 = av2[:, HEADS:]
    u = jnp.transpose(asrc3, (0, 2, 1)).reshape(G, HS)
    gmax = jnp.max(av2[:, :HEADS])
    shift = _leaky(adst2 + gmax)
    zeros_h = jnp.zeros((HEADS, HS), jnp.float32)
    zeros_g = jnp.zeros((G, HS), jnp.float32)
    left = jnp.concatenate([rexp, u, zeros_h], axis=0)
    right = jnp.concatenate([zeros_h, zeros_g, rexp], axis=0)
    rp = jnp.concatenate([left, right], axis=1)
    lhs = jnp.concatenate([adst2, gh1, shift], axis=1)
    rd = jax.lax.dot(lhs, rp)
    ex = jnp.exp(_leaky(rd[:, :HS]) - rd[:, HS:])

    xp3 = xp2.reshape(G, NP, H)
    ex3 = ex.reshape(G, NP, HS)
    outs = []
    for g in range(G):
        Eg = ex3[g] * A4_list[g]
        xstk = (xp3[g][None] * colmask).reshape(HS, H)
        res = jax.lax.dot(Eg, jnp.concatenate([xstk, sel_den], axis=1))
        outs.append(res[:, :H] / res[:, H:] + bias)
    return jnp.concatenate(outs, axis=0)


def _fused_kernel(feat_ref, ei_ref, wcat_ref, brow_ref,
                  w0_ref, att0_ref, b0_ref, w1_ref, att1_ref, b1_ref,
                  outh_ref, outv_ref):
    x2 = jax.lax.dot_general(
        feat_ref[...], wcat_ref[...], (((0,), (0,)), ((), ())))
    x2 = (x2.reshape(G, NP, H) + brow_ref[...][None]).reshape(G * NP, H)
    x2 = jnp.maximum(x2, 0.0)

    col_np = jax.lax.broadcasted_iota(jnp.int32, (NP, 1), 0)
    col_hs = jax.lax.broadcasted_iota(jnp.int32, (HS, 1), 0) % NP
    d_iota = jax.lax.broadcasted_iota(jnp.int32, (NP, HS), 0)
    s_iota = jax.lax.broadcasted_iota(jnp.int32, (NP, HS), 1) % NP
    eye4 = jnp.float32(d_iota == s_iota)

    A4_list = []
    for g in range(G):
        src = ei_ref[g][0:1, :]
        dst = ei_ref[g][1:2, :]
        oh_src4T = jnp.float32(src == col_hs)
        oh_dstT = jnp.float32(dst == col_np)
        A4 = jax.lax.dot_general(
            oh_dstT, oh_src4T, (((1,), (1,)), ((), ())))
        A4_list.append(A4 + eye4)

    hs_row = jax.lax.broadcasted_iota(jnp.int32, (HS, H), 0) // NP
    cc = jax.lax.broadcasted_iota(jnp.int32, (HS, H), 1) // DH
    sel_den = jnp.float32(hs_row == cc)
    rh = jax.lax.broadcasted_iota(jnp.int32, (HEADS, HS), 0)
    rc = jax.lax.broadcasted_iota(jnp.int32, (HEADS, HS), 1) // NP
    rexp = jnp.float32(rh == rc)
    gr = jax.lax.broadcasted_iota(jnp.int32, (G * NP, G), 0) // NP
    gc = jax.lax.broadcasted_iota(jnp.int32, (G * NP, G), 1)
    gh1 = jnp.float32(gr == gc)
    mh = jax.lax.broadcasted_iota(jnp.int32, (HEADS, 1, H), 0)
    mc = jax.lax.broadcasted_iota(jnp.int32, (HEADS, 1, H), 2) // DH
    colmask = jnp.float32(mh == mc)

    def layer(x2, w_ref, att_ref, b_ref):
        xp2 = jax.lax.dot(x2, w_ref[...])
        av2 = jax.lax.dot(xp2, att_ref[...])
        return _gat_block(xp2, av2, A4_list, sel_den, rexp, gh1, colmask,
                          b_ref[...])

    h1 = jnp.maximum(layer(x2, w0_ref, att0_ref, b0_ref), 0.0)
    h2 = jnp.maximum(layer(h1, w1_ref, att1_ref, b1_ref), 0.0)
    h3 = h2.reshape(G, NP, H)
    outh_ref[...] = h3[:, 0, :]
    outv_ref[...] = h3[:, N_PER - N_VAL:N_PER, :]


@jax.jit
def _run(feat, ei_t, wcat, brow, w0, att0, b0, w1, att1, b1):
    grid = (B // G,)
    return pl.pallas_call(
        _fused_kernel,
        grid=grid,
        in_specs=[
            pl.BlockSpec((9, G * NP), lambda i: (0, i)),
            pl.BlockSpec((G, 2, E), lambda i: (i, 0, 0)),
            pl.BlockSpec((9, H), lambda i: (0, 0)),
            pl.BlockSpec((NP, H), lambda i: (0, 0)),
            pl.BlockSpec((H, H), lambda i: (0, 0)),
            pl.BlockSpec((H, 2 * HEADS), lambda i: (0, 0)),
            pl.BlockSpec((1, H), lambda i: (0, 0)),
            pl.BlockSpec((H, H), lambda i: (0, 0)),
            pl.BlockSpec((H, 2 * HEADS), lambda i: (0, 0)),
            pl.BlockSpec((1, H), lambda i: (0, 0)),
        ],
        out_specs=[
            pl.BlockSpec((G, H), lambda i: (i, 0)),
            pl.BlockSpec((G, N_VAL, H), lambda i: (i, 0, 0)),
        ],
        out_shape=[
            jax.ShapeDtypeStruct((B, H), jnp.float32),
            jax.ShapeDtypeStruct((B, N_VAL, H), jnp.float32),
        ],
    )(feat, ei_t, wcat, brow, w0, att0, b0, w1, att1, b1)


def kernel(head_node, objective_nodes, value_nodes, edge_indices,
           W_head, b_head, W_obj, b_obj, W_val, b_val,
           W0, att_src0, att_dst0, bias0,
           W1, att_src1, att_dst1, bias1):
    f32 = jnp.float32
    feat3 = jnp.zeros((9, B, NP), f32)
    feat3 = feat3.at[0:2, :, 0].set(jnp.transpose(head_node))
    feat3 = feat3.at[2:4, :, 1:1 + N_OBJ].set(
        jnp.transpose(objective_nodes, (2, 0, 1)))
    feat3 = feat3.at[4:9, :, 1 + N_OBJ:N_PER].set(
        jnp.transpose(value_nodes, (2, 0, 1)))
    feat = feat3.reshape(9, B * NP)
    wcat = jnp.concatenate([W_head, W_obj, W_val], axis=0)
    brow = jnp.concatenate([
        b_head[None, :],
        jnp.tile(b_obj[None, :], (N_OBJ, 1)),
        jnp.tile(b_val[None, :], (N_VAL, 1)),
        jnp.zeros((NP - N_PER, H), f32),
    ], axis=0)
    att0 = jnp.zeros((H, 2 * HEADS), f32)
    att1 = jnp.zeros((H, 2 * HEADS), f32)
    for h in range(HEADS):
        att0 = att0.at[h * DH:(h + 1) * DH, h].set(att_src0[h])
        att0 = att0.at[h * DH:(h + 1) * DH, HEADS + h].set(att_dst0[h])
        att1 = att1.at[h * DH:(h + 1) * DH, h].set(att_src1[h])
        att1 = att1.at[h * DH:(h + 1) * DH, HEADS + h].set(att_dst1[h])
    outh, outv = _run(feat, edge_indices, wcat, brow,
                      W0, att0, bias0[None, :], W1, att1, bias1[None, :])
    return (outh, outv)


# speedup vs baseline: 551.6915x; 1.1054x over previous
import functools

import jax
import jax.numpy as jnp
import numpy as np
from jax.experimental import pallas as pl
from jax.experimental.pallas import tpu as pltpu

B = 4096
N_OBJ = 10
N_VAL = 50
N_PER = 61
NP = 64
E = 128
H = 128
HEADS = 4
DH = H // HEADS
HS = HEADS * NP
G = 16


def _leaky(x):
    return jnp.maximum(x, 0.2 * x)


def _gat_block(xp2, av2, A4_list, sel_den, rexp, gh1, colmask, bias):
    asrc3 = av2[:, :HEADS].reshape(G, NP, HEADS)
    adst2 = av2[:, HEADS:]
    u = jnp.transpose(asrc3, (0, 2, 1)).reshape(G, HS)
    c = _leaky(2.0 * jnp.max(av2))
    rp = jnp.concatenate([rexp, u], axis=0)
    lhs = jnp.concatenate([adst2, gh1], axis=1)
    rd = jax.lax.dot(lhs, rp)
    ex = jnp.exp(_leaky(rd) - c)

    xstk_all = (xp2.reshape(G, 1, NP, H) * colmask[None]).reshape(G, HS, H)
    ex3 = ex.reshape(G, NP, HS)
    outs = []
    for g in range(G):
        Eg = ex3[g] * A4_list[g]
        res = jax.lax.dot(
            Eg, jnp.concatenate([xstk_all[g], sel_den], axis=1))
        outs.append(res[:, :H] / res[:, H:] + bias)
    return jnp.concatenate(outs, axis=0)


def _fused_kernel(feat_ref, ei_ref, wcat_ref, brow_ref,
                  w0_ref, att0_ref, b0_ref, w1_ref, att1_ref, b1_ref,
                  outh_ref, outv_ref):
    x2 = jax.lax.dot_general(
        feat_ref[...], wcat_ref[...], (((0,), (0,)), ((), ())))
    x2 = (x2.reshape(G, NP, H) + brow_ref[...][None]).reshape(G * NP, H)
    x2 = jnp.maximum(x2, 0.0)

    col_np = jax.lax.broadcasted_iota(jnp.int32, (NP, 1), 0)
    col_hs = jax.lax.broadcasted_iota(jnp.int32, (HS, 1), 0) % NP
    d_iota = jax.lax.broadcasted_iota(jnp.int32, (NP, HS), 0)
    s_iota = jax.lax.broadcasted_iota(jnp.int32, (NP, HS), 1) % NP
    eye4 = jnp.float32(d_iota == s_iota)

    A4_list = []
    for g in range(G):
        src = ei_ref[g][0:1, :]
        dst = ei_ref[g][1:2, :]
        oh_src4T = jnp.float32(src == col_hs)
        oh_dstT = jnp.float32(dst == col_np)
        A4 = jax.lax.dot_general(
            oh_dstT, oh_src4T, (((1,), (1,)), ((), ())))
        A4_list.append(A4 + eye4)

    hs_row = jax.lax.broadcasted_iota(jnp.int32, (HS, H), 0) // NP
    cc = jax.lax.broadcasted_iota(jnp.int32, (HS, H), 1) // DH
    sel_den = jnp.float32(hs_row == cc)
    rh = jax.lax.broadcasted_iota(jnp.int32, (HEADS, HS), 0)
    rc = jax.lax.broadcasted_iota(jnp.int32, (HEADS, HS), 1) // NP
    rexp = jnp.float32(rh == rc)
    gr = jax.lax.broadcasted_iota(jnp.int32, (G * NP, G), 0) // NP
    gc = jax.lax.broadcasted_iota(jnp.int32, (G * NP, G), 1)
    gh1 = jnp.float32(gr == gc)
    mh = jax.lax.broadcasted_iota(jnp.int32, (HEADS, 1, H), 0)
    mc = jax.lax.broadcasted_iota(jnp.int32, (HEADS, 1, H), 2) // DH
    colmask = jnp.float32(mh == mc)

    def layer(x2, w_ref, att_ref, b_ref):
        xp2 = jax.lax.dot(x2, w_ref[...])
        av2 = jax.lax.dot(xp2, att_ref[...])
        return _gat_block(xp2, av2, A4_list, sel_den, rexp, gh1, colmask,
                          b_ref[...])

    h1 = jnp.maximum(layer(x2, w0_ref, att0_ref, b0_ref), 0.0)
    h2 = jnp.maximum(layer(h1, w1_ref, att1_ref, b1_ref), 0.0)
    h3 = h2.reshape(G, NP, H)
    outh_ref[...] = h3[:, 0, :]
    outv_ref[...] = h3[:, N_PER - N_VAL:N_PER, :]


@jax.jit
def _run(feat, ei_t, wcat, brow, w0, att0, b0, w1, att1, b1):
    grid = (B // G,)
    return pl.pallas_call(
        _fused_kernel,
        grid=grid,
        in_specs=[
            pl.BlockSpec((9, G * NP), lambda i: (0, i)),
            pl.BlockSpec((G, 2, E), lambda i: (i, 0, 0)),
            pl.BlockSpec((9, H), lambda i: (0, 0)),
            pl.BlockSpec((NP, H), lambda i: (0, 0)),
            pl.BlockSpec((H, H), lambda i: (0, 0)),
            pl.BlockSpec((H, 2 * HEADS), lambda i: (0, 0)),
            pl.BlockSpec((1, H), lambda i: (0, 0)),
            pl.BlockSpec((H, H), lambda i: (0, 0)),
            pl.BlockSpec((H, 2 * HEADS), lambda i: (0, 0)),
            pl.BlockSpec((1, H), lambda i: (0, 0)),
        ],
        out_specs=[
            pl.BlockSpec((G, H), lambda i: (i, 0)),
            pl.BlockSpec((G, N_VAL, H), lambda i: (i, 0, 0)),
        ],
        out_shape=[
            jax.ShapeDtypeStruct((B, H), jnp.float32),
            jax.ShapeDtypeStruct((B, N_VAL, H), jnp.float32),
        ],
    )(feat, ei_t, wcat, brow, w0, att0, b0, w1, att1, b1)


def kernel(head_node, objective_nodes, value_nodes, edge_indices,
           W_head, b_head, W_obj, b_obj, W_val, b_val,
           W0, att_src0, att_dst0, bias0,
           W1, att_src1, att_dst1, bias1):
    f32 = jnp.float32
    feat3 = jnp.zeros((9, B, NP), f32)
    feat3 = feat3.at[0:2, :, 0].set(jnp.transpose(head_node))
    feat3 = feat3.at[2:4, :, 1:1 + N_OBJ].set(
        jnp.transpose(objective_nodes, (2, 0, 1)))
    feat3 = feat3.at[4:9, :, 1 + N_OBJ:N_PER].set(
        jnp.transpose(value_nodes, (2, 0, 1)))
    feat = feat3.reshape(9, B * NP)
    wcat = jnp.concatenate([W_head, W_obj, W_val], axis=0)
    brow = jnp.concatenate([
        b_head[None, :],
        jnp.tile(b_obj[None, :], (N_OBJ, 1)),
        jnp.tile(b_val[None, :], (N_VAL, 1)),
        jnp.zeros((NP - N_PER, H), f32),
    ], axis=0)
    att0 = jnp.zeros((H, 2 * HEADS), f32)
    att1 = jnp.zeros((H, 2 * HEADS), f32)
    for h in range(HEADS):
        att0 = att0.at[h * DH:(h + 1) * DH, h].set(att_src0[h])
        att0 = att0.at[h * DH:(h + 1) * DH, HEADS + h].set(att_dst0[h])
        att1 = att1.at[h * DH:(h + 1) * DH, h].set(att_src1[h])
        att1 = att1.at[h * DH:(h + 1) * DH, HEADS + h].set(att_dst1[h])
    outh, outv = _run(feat, edge_indices, wcat, brow,
                      W0, att0, bias0[None, :], W1, att1, bias1[None, :])
    return (outh, outv)


# speedup vs baseline: 576.5911x; 1.0451x over previous
import functools

import jax
import jax.numpy as jnp
import numpy as np
from jax.experimental import pallas as pl
from jax.experimental.pallas import tpu as pltpu

B = 4096
N_OBJ = 10
N_VAL = 50
N_PER = 61
NP = 64
E = 128
H = 128
HEADS = 4
DH = H // HEADS
HS = HEADS * NP
AW = 2 * HEADS
G = 16


def _leaky(x):
    return jnp.maximum(x, 0.2 * x)


def _gat_block(xpav, A4_list, sel_den, rexp, gh1, colmask, bias):
    xp2 = xpav[:, :H]
    av2 = xpav[:, H:H + AW]
    asrc3 = av2[:, :HEADS].reshape(G, NP, HEADS)
    adst2 = av2[:, HEADS:]
    u = jnp.transpose(asrc3, (0, 2, 1)).reshape(G, HS)
    c = _leaky(2.0 * jnp.max(av2))
    rp = jnp.concatenate([rexp, u], axis=0)
    lhs = jnp.concatenate([adst2, gh1], axis=1)
    rd = jax.lax.dot(lhs, rp)
    ex = jnp.exp(_leaky(rd) - c)

    xstk_all = (xp2.reshape(G, 1, NP, H) * colmask[None]).reshape(G, HS, H)
    ex3 = ex.reshape(G, NP, HS)
    outs = []
    for g in range(G):
        Eg = ex3[g] * A4_list[g]
        res = jax.lax.dot(
            Eg, jnp.concatenate([xstk_all[g], sel_den], axis=1))
        outs.append(res[:, :H] / res[:, H:] + bias)
    return jnp.concatenate(outs, axis=0)


def _fused_kernel(feat_ref, ei_ref, wcat_ref, brow_ref,
                  wa0_ref, b0_ref, wa1_ref, b1_ref,
                  sel_den_ref, rexp_ref, gh1_ref, colmask_ref, eye4_ref,
                  outh_ref, outv_ref):
    x2 = jax.lax.dot_general(
        feat_ref[...], wcat_ref[...], (((0,), (0,)), ((), ())))
    x2 = (x2.reshape(G, NP, H) + brow_ref[...][None]).reshape(G * NP, H)
    x2 = jnp.maximum(x2, 0.0)

    col_np = jax.lax.broadcasted_iota(jnp.int32, (NP, 1), 0)
    col_hs = jax.lax.broadcasted_iota(jnp.int32, (HS, 1), 0) % NP

    A4_list = []
    for g in range(G):
        src = ei_ref[g][0:1, :]
        dst = ei_ref[g][1:2, :]
        oh_src4T = jnp.float32(src == col_hs)
        oh_dstT = jnp.float32(dst == col_np)
        A4 = jax.lax.dot_general(
            oh_dstT, oh_src4T, (((1,), (1,)), ((), ())))
        A4_list.append(A4 + eye4_ref[...])

    sel_den = sel_den_ref[...]
    rexp = rexp_ref[...]
    gh1 = gh1_ref[...]
    colmask = colmask_ref[...][:, None, :]

    def layer(x2, wa_ref, b_ref):
        xpav = jax.lax.dot(x2, wa_ref[...])
        return _gat_block(xpav, A4_list, sel_den, rexp, gh1, colmask,
                          b_ref[...])

    h1 = jnp.maximum(layer(x2, wa0_ref, b0_ref), 0.0)
    h2 = jnp.maximum(layer(h1, wa1_ref, b1_ref), 0.0)
    h3 = h2.reshape(G, NP, H)
    outh_ref[...] = h3[:, 0, :]
    outv_ref[...] = h3[:, N_PER - N_VAL:N_PER, :]


@jax.jit
def _run(feat, ei, wcat, brow, wa0, b0, wa1, b1,
         sel_den, rexp, gh1, colmask, eye4):
    grid = (B // G,)
    full = lambda *s: pl.BlockSpec(s, lambda i: tuple(0 for _ in s))
    return pl.pallas_call(
        _fused_kernel,
        grid=grid,
        in_specs=[
            pl.BlockSpec((9, G * NP), lambda i: (0, i)),
            pl.BlockSpec((G, 2, E), lambda i: (i, 0, 0)),
            full(9, H),
            full(NP, H),
            full(H, H + AW),
            full(1, H),
            full(H, H + AW),
            full(1, H),
            full(HS, H),
            full(HEADS, HS),
            full(G * NP, G),
            full(HEADS, H),
            full(NP, HS),
        ],
        out_specs=[
            pl.BlockSpec((G, H), lambda i: (i, 0)),
            pl.BlockSpec((G, N_VAL, H), lambda i: (i, 0, 0)),
        ],
        out_shape=[
            jax.ShapeDtypeStruct((B, H), jnp.float32),
            jax.ShapeDtypeStruct((B, N_VAL, H), jnp.float32),
        ],
    )(feat, ei, wcat, brow, wa0, b0, wa1, b1,
      sel_den, rexp, gh1, colmask, eye4)


def kernel(head_node, objective_nodes, value_nodes, edge_indices,
           W_head, b_head, W_obj, b_obj, W_val, b_val,
           W0, att_src0, att_dst0, bias0,
           W1, att_src1, att_dst1, bias1):
    f32 = jnp.float32
    feat3 = jnp.zeros((9, B, NP), f32)
    feat3 = feat3.at[0:2, :, 0].set(jnp.transpose(head_node))
    feat3 = feat3.at[2:4, :, 1:1 + N_OBJ].set(
        jnp.transpose(objective_nodes, (2, 0, 1)))
    feat3 = feat3.at[4:9, :, 1 + N_OBJ:N_PER].set(
        jnp.transpose(value_nodes, (2, 0, 1)))
    feat = feat3.reshape(9, B * NP)
    wcat = jnp.concatenate([W_head, W_obj, W_val], axis=0)
    brow = jnp.concatenate([
        b_head[None, :],
        jnp.tile(b_obj[None, :], (N_OBJ, 1)),
        jnp.tile(b_val[None, :], (N_VAL, 1)),
        jnp.zeros((NP - N_PER, H), f32),
    ], axis=0)
    att0 = jnp.zeros((H, AW), f32)
    att1 = jnp.zeros((H, AW), f32)
    for h in range(HEADS):
        att0 = att0.at[h * DH:(h + 1) * DH, h].set(att_src0[h])
        att0 = att0.at[h * DH:(h + 1) * DH, HEADS + h].set(att_dst0[h])
        att1 = att1.at[h * DH:(h + 1) * DH, h].set(att_src1[h])
        att1 = att1.at[h * DH:(h + 1) * DH, HEADS + h].set(att_dst1[h])
    wa0 = jnp.concatenate([W0, W0 @ att0], axis=1)
    wa1 = jnp.concatenate([W1, W1 @ att1], axis=1)
    ii = np.arange(HS)
    sel_den = jnp.asarray((ii[:, None] // NP) == (np.arange(H)[None] // DH),
                          f32)
    rexp = jnp.asarray(np.arange(HEADS)[:, None] == (ii[None] // NP), f32)
    gh1 = jnp.asarray((np.arange(G * NP)[:, None] // NP)
                      == np.arange(G)[None], f32)
    colmask = jnp.asarray(np.arange(HEADS)[:, None]
                          == (np.arange(H)[None] // DH), f32)
    eye4 = jnp.asarray(np.arange(NP)[:, None] == (ii[None] % NP), f32)
    outh, outv = _run(feat, edge_indices, wcat, brow,
                      wa0, bias0[None, :], wa1, bias1[None, :],
                      sel_den, rexp, gh1, colmask, eye4)
    return (outh, outv)


# speedup vs baseline: 646.3288x; 1.1209x over previous
import functools

import jax
import jax.numpy as jnp
import numpy as np
from jax.experimental import pallas as pl
from jax.experimental.pallas import tpu as pltpu

B = 4096
N_OBJ = 10
N_VAL = 50
N_PER = 61
NP = 64
E = 128
H = 128
HEADS = 4
DH = H // HEADS
HS = HEADS * NP
AW = 2 * HEADS
G = 32


def _leaky(x):
    return jnp.maximum(x, 0.2 * x)


def _gat_block(xpav, A4_list, sel_den, rexp, gh1, colmask, bias):
    xp2 = xpav[:, :H]
    av2 = xpav[:, H:H + AW]
    asrc3 = av2[:, :HEADS].reshape(G, NP, HEADS)
    adst2 = av2[:, HEADS:]
    u = jnp.transpose(asrc3, (0, 2, 1)).reshape(G, HS)
    c = _leaky(2.0 * jnp.max(av2))
    rp = jnp.concatenate([rexp, u], axis=0)
    lhs = jnp.concatenate([adst2, gh1], axis=1)
    rd = jax.lax.dot(lhs, rp)
    ex = jnp.exp(_leaky(rd) - c)

    xstk_all = (xp2.reshape(G, 1, NP, H) * colmask[None]).reshape(G, HS, H)
    ex3 = ex.reshape(G, NP, HS)
    outs = []
    for g in range(G):
        Eg = ex3[g] * A4_list[g]
        res = jax.lax.dot(
            Eg, jnp.concatenate([xstk_all[g], sel_den], axis=1))
        outs.append(res[:, :H] / res[:, H:] + bias)
    return jnp.concatenate(outs, axis=0)


def _fused_kernel(feat_ref, ei_ref, wcat_ref, brow_ref,
                  wa0_ref, b0_ref, wa1_ref, b1_ref,
                  sel_den_ref, rexp_ref, gh1_ref, colmask_ref, eye4_ref,
                  outh_ref, outv_ref):
    x2 = jax.lax.dot_general(
        feat_ref[...], wcat_ref[...], (((0,), (0,)), ((), ())))
    x2 = (x2.reshape(G, NP, H) + brow_ref[...][None]).reshape(G * NP, H)
    x2 = jnp.maximum(x2, 0.0)

    col_np = jax.lax.broadcasted_iota(jnp.int32, (NP, 1), 0)
    col_hs = jax.lax.broadcasted_iota(jnp.int32, (HS, 1), 0) % NP

    A4_list = []
    for g in range(G):
        src = ei_ref[g][0:1, :]
        dst = ei_ref[g][1:2, :]
        oh_src4T = jnp.float32(src == col_hs)
        oh_dstT = jnp.float32(dst == col_np)
        A4 = jax.lax.dot_general(
            oh_dstT, oh_src4T, (((1,), (1,)), ((), ())))
        A4_list.append(A4 + eye4_ref[...])

    sel_den = sel_den_ref[...]
    rexp = rexp_ref[...]
    gh1 = gh1_ref[...]
    colmask = colmask_ref[...][:, None, :]

    def layer(x2, wa_ref, b_ref):
        xpav = jax.lax.dot(x2, wa_ref[...])
        return _gat_block(xpav, A4_list, sel_den, rexp, gh1, colmask,
                          b_ref[...])

    h1 = jnp.maximum(layer(x2, wa0_ref, b0_ref), 0.0)
    h2 = jnp.maximum(layer(h1, wa1_ref, b1_ref), 0.0)
    h3 = h2.reshape(G, NP, H)
    outh_ref[...] = h3[:, 0, :]
    outv_ref[...] = h3[:, N_PER - N_VAL:N_PER, :]


@jax.jit
def _run(feat, ei, wcat, brow, wa0, b0, wa1, b1,
         sel_den, rexp, gh1, colmask, eye4):
    grid = (B // G,)
    full = lambda *s: pl.BlockSpec(s, lambda i: tuple(0 for _ in s))
    return pl.pallas_call(
        _fused_kernel,
        grid=grid,
        in_specs=[
            pl.BlockSpec((9, G * NP), lambda i: (0, i)),
            pl.BlockSpec((G, 2, E), lambda i: (i, 0, 0)),
            full(9, H),
            full(NP, H),
            full(H, H + AW),
            full(1, H),
            full(H, H + AW),
            full(1, H),
            full(HS, H),
            full(HEADS, HS),
            full(G * NP, G),
            full(HEADS, H),
            full(NP, HS),
        ],
        out_specs=[
            pl.BlockSpec((G, H), lambda i: (i, 0)),
            pl.BlockSpec((G, N_VAL, H), lambda i: (i, 0, 0)),
        ],
        out_shape=[
            jax.ShapeDtypeStruct((B, H), jnp.float32),
            jax.ShapeDtypeStruct((B, N_VAL, H), jnp.float32),
        ],
    )(feat, ei, wcat, brow, wa0, b0, wa1, b1,
      sel_den, rexp, gh1, colmask, eye4)


def kernel(head_node, objective_nodes, value_nodes, edge_indices,
           W_head, b_head, W_obj, b_obj, W_val, b_val,
           W0, att_src0, att_dst0, bias0,
           W1, att_src1, att_dst1, bias1):
    f32 = jnp.float32
    feat3 = jnp.zeros((9, B, NP), f32)
    feat3 = feat3.at[0:2, :, 0].set(jnp.transpose(head_node))
    feat3 = feat3.at[2:4, :, 1:1 + N_OBJ].set(
        jnp.transpose(objective_nodes, (2, 0, 1)))
    feat3 = feat3.at[4:9, :, 1 + N_OBJ:N_PER].set(
        jnp.transpose(value_nodes, (2, 0, 1)))
    feat = feat3.reshape(9, B * NP)
    wcat = jnp.concatenate([W_head, W_obj, W_val], axis=0)
    brow = jnp.concatenate([
        b_head[None, :],
        jnp.tile(b_obj[None, :], (N_OBJ, 1)),
        jnp.tile(b_val[None, :], (N_VAL, 1)),
        jnp.zeros((NP - N_PER, H), f32),
    ], axis=0)
    att0 = jnp.zeros((H, AW), f32)
    att1 = jnp.zeros((H, AW), f32)
    for h in range(HEADS):
        att0 = att0.at[h * DH:(h + 1) * DH, h].set(att_src0[h])
        att0 = att0.at[h * DH:(h + 1) * DH, HEADS + h].set(att_dst0[h])
        att1 = att1.at[h * DH:(h + 1) * DH, h].set(att_src1[h])
        att1 = att1.at[h * DH:(h + 1) * DH, HEADS + h].set(att_dst1[h])
    wa0 = jnp.concatenate([W0, W0 @ att0], axis=1)
    wa1 = jnp.concatenate([W1, W1 @ att1], axis=1)
    ii = np.arange(HS)
    sel_den = jnp.asarray((ii[:, None] // NP) == (np.arange(H)[None] // DH),
                          f32)
    rexp = jnp.asarray(np.arange(HEADS)[:, None] == (ii[None] // NP), f32)
    gh1 = jnp.asarray((np.arange(G * NP)[:, None] // NP)
                      == np.arange(G)[None], f32)
    colmask = jnp.asarray(np.arange(HEADS)[:, None]
                          == (np.arange(H)[None] // DH), f32)
    eye4 = jnp.asarray(np.arange(NP)[:, None] == (ii[None] % NP), f32)
    outh, outv = _run(feat, edge_indices, wcat, brow,
                      wa0, bias0[None, :], wa1, bias1[None, :],
                      sel_den, rexp, gh1, colmask, eye4)
    return (outh, outv)


# speedup vs baseline: 703.8010x; 1.0889x over previous
import functools

import jax
import jax.numpy as jnp
import numpy as np
from jax.experimental import pallas as pl
from jax.experimental.pallas import tpu as pltpu

B = 4096
N_OBJ = 10
N_VAL = 50
N_PER = 61
NP = 64
E = 128
H = 128
HEADS = 4
DH = H // HEADS
HS = HEADS * NP
AW = 2 * HEADS
G = 64


def _leaky(x):
    return jnp.maximum(x, 0.2 * x)


def _gat_block(xpav, A4_list, sel_den, rexp, gh1, colmask, bias):
    xp2 = xpav[:, :H]
    av2 = xpav[:, H:H + AW]
    asrc3 = av2[:, :HEADS].reshape(G, NP, HEADS)
    adst2 = av2[:, HEADS:]
    u = jnp.transpose(asrc3, (0, 2, 1)).reshape(G, HS)
    c = _leaky(2.0 * jnp.max(av2))
    rp = jnp.concatenate([rexp, u], axis=0)
    lhs = jnp.concatenate([adst2, gh1], axis=1)
    rd = jax.lax.dot(lhs, rp)
    ex = jnp.exp(_leaky(rd) - c)

    xstk_all = (xp2.reshape(G, 1, NP, H) * colmask[None]).reshape(G, HS, H)
    ex3 = ex.reshape(G, NP, HS)
    outs = []
    for g in range(G):
        Eg = ex3[g] * A4_list[g]
        res = jax.lax.dot(
            Eg, jnp.concatenate([xstk_all[g], sel_den], axis=1))
        outs.append(res[:, :H] / res[:, H:] + bias)
    return jnp.concatenate(outs, axis=0)


def _fused_kernel(feat_ref, ei_ref, wcat_ref, brow_ref,
                  wa0_ref, b0_ref, wa1_ref, b1_ref,
                  sel_den_ref, rexp_ref, gh1_ref, colmask_ref, eye4_ref,
                  outh_ref, outv_ref):
    x2 = jax.lax.dot_general(
        feat_ref[...], wcat_ref[...], (((0,), (0,)), ((), ())))
    x2 = (x2.reshape(G, NP, H) + brow_ref[...][None]).reshape(G * NP, H)
    x2 = jnp.maximum(x2, 0.0)

    col_np = jax.lax.broadcasted_iota(jnp.int32, (NP, 1), 0)
    col_hs = jax.lax.broadcasted_iota(jnp.int32, (HS, 1), 0) % NP

    A4_list = []
    for g in range(G):
        src = ei_ref[g][0:1, :]
        dst = ei_ref[g][1:2, :]
        oh_src4T = jnp.float32(src == col_hs)
        oh_dstT = jnp.float32(dst == col_np)
        A4 = jax.lax.dot_general(
            oh_dstT, oh_src4T, (((1,), (1,)), ((), ())))
        A4_list.append(A4 + eye4_ref[...])

    sel_den = sel_den_ref[...]
    rexp = rexp_ref[...]
    gh1 = gh1_ref[...]
    colmask = colmask_ref[...][:, None, :]

    def layer(x2, wa_ref, b_ref):
        xpav = jax.lax.dot(x2, wa_ref[...])
        return _gat_block(xpav, A4_list, sel_den, rexp, gh1, colmask,
                          b_ref[...])

    h1 = jnp.maximum(layer(x2, wa0_ref, b0_ref), 0.0)
    h2 = jnp.maximum(layer(h1, wa1_ref, b1_ref), 0.0)
    h3 = h2.reshape(G, NP, H)
    outh_ref[...] = h3[:, 0, :]
    outv_ref[...] = h3[:, N_PER - N_VAL:N_PER, :]


@jax.jit
def _run(feat, ei, wcat, brow, wa0, b0, wa1, b1,
         sel_den, rexp, gh1, colmask, eye4):
    grid = (B // G,)
    full = lambda *s: pl.BlockSpec(s, lambda i: tuple(0 for _ in s))
    return pl.pallas_call(
        _fused_kernel,
        grid=grid,
        in_specs=[
            pl.BlockSpec((9, G * NP), lambda i: (0, i)),
            pl.BlockSpec((G, 2, E), lambda i: (i, 0, 0)),
            full(9, H),
            full(NP, H),
            full(H, H + AW),
            full(1, H),
            full(H, H + AW),
            full(1, H),
            full(HS, H),
            full(HEADS, HS),
            full(G * NP, G),
            full(HEADS, H),
            full(NP, HS),
        ],
        out_specs=[
            pl.BlockSpec((G, H), lambda i: (i, 0)),
            pl.BlockSpec((G, N_VAL, H), lambda i: (i, 0, 0)),
        ],
        out_shape=[
            jax.ShapeDtypeStruct((B, H), jnp.float32),
            jax.ShapeDtypeStruct((B, N_VAL, H), jnp.float32),
        ],
    )(feat, ei, wcat, brow, wa0, b0, wa1, b1,
      sel_den, rexp, gh1, colmask, eye4)


def kernel(head_node, objective_nodes, value_nodes, edge_indices,
           W_head, b_head, W_obj, b_obj, W_val, b_val,
           W0, att_src0, att_dst0, bias0,
           W1, att_src1, att_dst1, bias1):
    f32 = jnp.float32
    feat3 = jnp.zeros((9, B, NP), f32)
    feat3 = feat3.at[0:2, :, 0].set(jnp.transpose(head_node))
    feat3 = feat3.at[2:4, :, 1:1 + N_OBJ].set(
        jnp.transpose(objective_nodes, (2, 0, 1)))
    feat3 = feat3.at[4:9, :, 1 + N_OBJ:N_PER].set(
        jnp.transpose(value_nodes, (2, 0, 1)))
    feat = feat3.reshape(9, B * NP)
    wcat = jnp.concatenate([W_head, W_obj, W_val], axis=0)
    brow = jnp.concatenate([
        b_head[None, :],
        jnp.tile(b_obj[None, :], (N_OBJ, 1)),
        jnp.tile(b_val[None, :], (N_VAL, 1)),
        jnp.zeros((NP - N_PER, H), f32),
    ], axis=0)
    att0 = jnp.zeros((H, AW), f32)
    att1 = jnp.zeros((H, AW), f32)
    for h in range(HEADS):
        att0 = att0.at[h * DH:(h + 1) * DH, h].set(att_src0[h])
        att0 = att0.at[h * DH:(h + 1) * DH, HEADS + h].set(att_dst0[h])
        att1 = att1.at[h * DH:(h + 1) * DH, h].set(att_src1[h])
        att1 = att1.at[h * DH:(h + 1) * DH, HEADS + h].set(att_dst1[h])
    wa0 = jnp.concatenate([W0, W0 @ att0], axis=1)
    wa1 = jnp.concatenate([W1, W1 @ att1], axis=1)
    ii = np.arange(HS)
    sel_den = jnp.asarray((ii[:, None] // NP) == (np.arange(H)[None] // DH),
                          f32)
    rexp = jnp.asarray(np.arange(HEADS)[:, None] == (ii[None] // NP), f32)
    gh1 = jnp.asarray((np.arange(G * NP)[:, None] // NP)
                      == np.arange(G)[None], f32)
    colmask = jnp.asarray(np.arange(HEADS)[:, None]
                          == (np.arange(H)[None] // DH), f32)
    eye4 = jnp.asarray(np.arange(NP)[:, None] == (ii[None] % NP), f32)
    outh, outv = _run(feat, edge_indices, wcat, brow,
                      wa0, bias0[None, :], wa1, bias1[None, :],
                      sel_den, rexp, gh1, colmask, eye4)
    return (outh, outv)


# speedup vs baseline: 723.3467x; 1.0278x over previous
import functools

import jax
import jax.numpy as jnp
import numpy as np
from jax.experimental import pallas as pl
from jax.experimental.pallas import tpu as pltpu

B = 4096
N_OBJ = 10
N_VAL = 50
N_PER = 61
NP = 64
E = 128
H = 128
HEADS = 4
DH = H // HEADS
HS = HEADS * NP
AW = 2 * HEADS
G = 128


def _leaky(x):
    return jnp.maximum(x, 0.2 * x)


def _gat_block(xpav, A4_list, sel_den, rexp, gh1, colmask, bias):
    xp2 = xpav[:, :H]
    av2 = xpav[:, H:H + AW]
    asrc3 = av2[:, :HEADS].reshape(G, NP, HEADS)
    adst2 = av2[:, HEADS:]
    u = jnp.transpose(asrc3, (0, 2, 1)).reshape(G, HS)
    c = _leaky(2.0 * jnp.max(av2))
    rp = jnp.concatenate([rexp, u], axis=0)
    lhs = jnp.concatenate([adst2, gh1], axis=1)
    rd = jax.lax.dot(lhs, rp)
    ex = jnp.exp(_leaky(rd) - c)

    xstk_all = (xp2.reshape(G, 1, NP, H) * colmask[None]).reshape(G, HS, H)
    ex3 = ex.reshape(G, NP, HS)
    outs = []
    for g in range(G):
        Eg = ex3[g] * A4_list[g]
        res = jax.lax.dot(
            Eg, jnp.concatenate([xstk_all[g], sel_den], axis=1))
        outs.append(res[:, :H] / res[:, H:] + bias)
    return jnp.concatenate(outs, axis=0)


def _fused_kernel(feat_ref, ei_ref, wcat_ref, brow_ref,
                  wa0_ref, b0_ref, wa1_ref, b1_ref,
                  sel_den_ref, rexp_ref, gh1_ref, colmask_ref, eye4_ref,
                  outh_ref, outv_ref):
    x2 = jax.lax.dot_general(
        feat_ref[...], wcat_ref[...], (((0,), (0,)), ((), ())))
    x2 = (x2.reshape(G, NP, H) + brow_ref[...][None]).reshape(G * NP, H)
    x2 = jnp.maximum(x2, 0.0)

    col_np = jax.lax.broadcasted_iota(jnp.int32, (NP, 1), 0)
    col_hs = jax.lax.broadcasted_iota(jnp.int32, (HS, 1), 0) % NP

    A4_list = []
    for g in range(G):
        src = ei_ref[g][0:1, :]
        dst = ei_ref[g][1:2, :]
        oh_src4T = jnp.float32(src == col_hs)
        oh_dstT = jnp.float32(dst == col_np)
        A4 = jax.lax.dot_general(
            oh_dstT, oh_src4T, (((1,), (1,)), ((), ())))
        A4_list.append(A4 + eye4_ref[...])

    sel_den = sel_den_ref[...]
    rexp = rexp_ref[...]
    gh1 = gh1_ref[...]
    colmask = colmask_ref[...][:, None, :]

    def layer(x2, wa_ref, b_ref):
        xpav = jax.lax.dot(x2, wa_ref[...])
        return _gat_block(xpav, A4_list, sel_den, rexp, gh1, colmask,
                          b_ref[...])

    h1 = jnp.maximum(layer(x2, wa0_ref, b0_ref), 0.0)
    h2 = jnp.maximum(layer(h1, wa1_ref, b1_ref), 0.0)
    h3 = h2.reshape(G, NP, H)
    outh_ref[...] = h3[:, 0, :]
    outv_ref[...] = h3[:, N_PER - N_VAL:N_PER, :]


@jax.jit
def _run(feat, ei, wcat, brow, wa0, b0, wa1, b1,
         sel_den, rexp, gh1, colmask, eye4):
    grid = (B // G,)
    full = lambda *s: pl.BlockSpec(s, lambda i: tuple(0 for _ in s))
    return pl.pallas_call(
        _fused_kernel,
        grid=grid,
        in_specs=[
            pl.BlockSpec((9, G * NP), lambda i: (0, i)),
            pl.BlockSpec((G, 2, E), lambda i: (i, 0, 0)),
            full(9, H),
            full(NP, H),
            full(H, H + AW),
            full(1, H),
            full(H, H + AW),
            full(1, H),
            full(HS, H),
            full(HEADS, HS),
            full(G * NP, G),
            full(HEADS, H),
            full(NP, HS),
        ],
        out_specs=[
            pl.BlockSpec((G, H), lambda i: (i, 0)),
            pl.BlockSpec((G, N_VAL, H), lambda i: (i, 0, 0)),
        ],
        out_shape=[
            jax.ShapeDtypeStruct((B, H), jnp.float32),
            jax.ShapeDtypeStruct((B, N_VAL, H), jnp.float32),
        ],
    )(feat, ei, wcat, brow, wa0, b0, wa1, b1,
      sel_den, rexp, gh1, colmask, eye4)


def kernel(head_node, objective_nodes, value_nodes, edge_indices,
           W_head, b_head, W_obj, b_obj, W_val, b_val,
           W0, att_src0, att_dst0, bias0,
           W1, att_src1, att_dst1, bias1):
    f32 = jnp.float32
    feat3 = jnp.zeros((9, B, NP), f32)
    feat3 = feat3.at[0:2, :, 0].set(jnp.transpose(head_node))
    feat3 = feat3.at[2:4, :, 1:1 + N_OBJ].set(
        jnp.transpose(objective_nodes, (2, 0, 1)))
    feat3 = feat3.at[4:9, :, 1 + N_OBJ:N_PER].set(
        jnp.transpose(value_nodes, (2, 0, 1)))
    feat = feat3.reshape(9, B * NP)
    wcat = jnp.concatenate([W_head, W_obj, W_val], axis=0)
    brow = jnp.concatenate([
        b_head[None, :],
        jnp.tile(b_obj[None, :], (N_OBJ, 1)),
        jnp.tile(b_val[None, :], (N_VAL, 1)),
        jnp.zeros((NP - N_PER, H), f32),
    ], axis=0)
    att0 = jnp.zeros((H, AW), f32)
    att1 = jnp.zeros((H, AW), f32)
    for h in range(HEADS):
        att0 = att0.at[h * DH:(h + 1) * DH, h].set(att_src0[h])
        att0 = att0.at[h * DH:(h + 1) * DH, HEADS + h].set(att_dst0[h])
        att1 = att1.at[h * DH:(h + 1) * DH, h].set(att_src1[h])
        att1 = att1.at[h * DH:(h + 1) * DH, HEADS + h].set(att_dst1[h])
    wa0 = jnp.concatenate([W0, W0 @ att0], axis=1)
    wa1 = jnp.concatenate([W1, W1 @ att1], axis=1)
    ii = np.arange(HS)
    sel_den = jnp.asarray((ii[:, None] // NP) == (np.arange(H)[None] // DH),
                          f32)
    rexp = jnp.asarray(np.arange(HEADS)[:, None] == (ii[None] // NP), f32)
    gh1 = jnp.asarray((np.arange(G * NP)[:, None] // NP)
                      == np.arange(G)[None], f32)
    colmask = jnp.asarray(np.arange(HEADS)[:, None]
                          == (np.arange(H)[None] // DH), f32)
    eye4 = jnp.asarray(np.arange(NP)[:, None] == (ii[None] % NP), f32)
    outh, outv = _run(feat, edge_indices, wcat, brow,
                      wa0, bias0[None, :], wa1, bias1[None, :],
                      sel_den, rexp, gh1, colmask, eye4)
    return (outh, outv)
